# baseline (speedup 1.0000x reference)
import functools

import jax
import jax.numpy as jnp
from jax import lax
from jax.experimental import pallas as pl
from jax.experimental.pallas import tpu as pltpu
from jax.experimental.pallas import tpu_sc as plsc

BATCH = 4
C = 19
TPAD = 128
OPAD = 32
PIX = 512 * 512
N = BATCH * PIX
THR = 2.3

BLK = 8192
NB = N // BLK
NBB = PIX // BLK
NG = BLK // 16

NW = 32
CHUNK = N // NW
GROUP = 128
NGRP = CHUNK // GROUP
SUPER = 2
NSUP = NGRP // SUPER
DUMMY = N


def _tree_sum(vals):
    vals = list(vals)
    n = 16
    while n >= 1:
        for i in range(n):
            if i + n < len(vals):
                vals[i] = vals[i] + vals[i + n]
        vals = vals[:n]
        n //= 2
    return vals[0]


def _tree_max(vals):
    vals = list(vals)
    n = 16
    while n >= 1:
        for i in range(n):
            if i + n < len(vals):
                vals[i] = jnp.maximum(vals[i], vals[i + n])
        vals = vals[:n]
        n //= 2
    return vals[0]


def tc_entropy_body(t_ref, lo_ref, table_ref, mask_ref, pref_ref, gpref_ref,
                    acc_ref):
    i = pl.program_id(0)

    @pl.when(i == 0)
    def _():
        acc_ref[0] = 0

    x = lo_ref[0]
    planes = [x[c] for c in range(C)]
    m = _tree_max(planes)
    e = jnp.exp(x - m[None, :])
    s = _tree_sum([e[c] for c in range(C)])
    p = e / s[None, :]
    t = (-p) * jnp.log(p)
    h = _tree_sum([t[c] for c in range(C)])
    msk = (h < THR).astype(jnp.int32)
    scaled = x / t_ref[0]

    pref_ref[0] = acc_ref[0].reshape(1, 1)

    mf = (h < THR).astype(jnp.float32).reshape(64, 128)
    li = lax.broadcasted_iota(jnp.int32, (128, 8), 0)
    lj = lax.broadcasted_iota(jnp.int32, (128, 8), 1)
    P = jnp.where(li // 16 == lj, 1.0, 0.0)
    g2 = jnp.dot(mf, P, preferred_element_type=jnp.float32)
    t8r = lax.broadcasted_iota(jnp.int32, (8, 8), 0)
    t8c = lax.broadcasted_iota(jnp.int32, (8, 8), 1)
    T8 = jnp.where(t8r < t8c, 1.0, 0.0)
    g2ex = jnp.dot(g2, T8, preferred_element_type=jnp.float32)
    rt = jnp.sum(g2, axis=1, keepdims=True)
    lr = lax.broadcasted_iota(jnp.int32, (64, 64), 0)
    lc = lax.broadcasted_iota(jnp.int32, (64, 64), 1)
    L = jnp.where(lc < lr, 1.0, 0.0)
    ro2 = jnp.dot(L, rt, preferred_element_type=jnp.float32)
    accf = acc_ref[0].astype(jnp.float32)
    gpref_ref[0] = (g2ex + ro2 + accf).astype(jnp.int32)

    @pl.when(i < NB)
    def _():
        table_ref[0] = jnp.concatenate(
            [scaled.T, jnp.ones((BLK, TPAD - C), jnp.float32)], axis=1)
        mask_ref[0, 0] = msk
        acc_ref[0] = acc_ref[0] + jnp.sum(msk)

    @pl.when(i == NB)
    def _():
        table_ref[0] = jnp.ones((BLK, TPAD), jnp.float32)
        mask_ref[0, 0] = jnp.zeros((BLK,), jnp.int32)


def make_tc_call(interpret=False):
    return pl.pallas_call(
        tc_entropy_body,
        grid=(NB + 1,),
        in_specs=[
            pl.BlockSpec(memory_space=pltpu.SMEM),
            pl.BlockSpec(
                (1, C, BLK),
                lambda i: (jnp.minimum(i, NB - 1) // NBB, 0,
                           jnp.minimum(i, NB - 1) % NBB),
            ),
        ],
        out_specs=[
            pl.BlockSpec((1, BLK, TPAD), lambda i: (i, 0, 0)),
            pl.BlockSpec((1, 1, BLK), lambda i: (i, 0, 0)),
            pl.BlockSpec((1, 1, 1), lambda i: (i, 0, 0)),
            pl.BlockSpec((1, 64, 8), lambda i: (i, 0, 0)),
        ],
        out_shape=[
            jax.ShapeDtypeStruct((NB + 1, BLK, TPAD), jnp.float32),
            jax.ShapeDtypeStruct((NB + 1, 1, BLK), jnp.int32),
            jax.ShapeDtypeStruct((NB + 1, 1, 1), jnp.int32),
            jax.ShapeDtypeStruct((NB + 1, 64, 8), jnp.int32),
        ],
        scratch_shapes=[pltpu.SMEM((1,), jnp.int32)],
        interpret=interpret,
    )


def sc_compact_body(table_hbm, mask_hbm, prefix_hbm, gpref_hbm, out_hbm,
                    prefsmem, gprefsmem, maskbuf, gprefv, idxbuf, rowbuf,
                    pack32, gsem):
    wid = lax.axis_index("s") * 2 + lax.axis_index("c")
    base = wid * CHUNK
    lane = lax.iota(jnp.int32, 16)
    dummy16 = jnp.full((16,), DUMMY, jnp.int32)

    pltpu.sync_copy(prefix_hbm, maskbuf.at[pl.ds(0, NB + 16)])
    for k in range((NB + 16) // 16):
        v = maskbuf[pl.ds(16 * k, 16)]
        for j in range(16):
            prefsmem[16 * k + j] = v[j]

    def prefill(i, _):
        idxbuf[pl.ds(16 * i, 16)] = dummy16
        return 0
    lax.fori_loop(0, (CHUNK + 32) // 16, prefill, 0)

    ktot = prefsmem[NB]
    kend = jnp.clip(ktot - base, 0, CHUNK)

    def blk_body(b, _):
        pb = prefsmem[b]
        pb1 = prefsmem[b + 1]
        live = jnp.logical_and(
            jnp.logical_and(pb1 > base, pb < base + CHUNK),
            pb1 > pb)

        @pl.when(live)
        def _():
            pltpu.sync_copy(mask_hbm.at[pl.ds(b * BLK, BLK)], maskbuf)
            pltpu.sync_copy(gpref_hbm.at[pl.ds(b * NG, NG)], gprefv)
            for k in range(NG // 16):
                v = gprefv[pl.ds(16 * k, 16)]
                for j in range(16):
                    gprefsmem[16 * k + j] = v[j]
            gbase = b * BLK

            def step(g, _):
                wp = gprefsmem[g] - base
                off = jnp.clip(wp, 0, kend)
                skip = off - wp
                mv = maskbuf[pl.ds(16 * g, 16)]
                mb = mv > 0
                one = jnp.where(mb, 1, 0)
                x = one
                for d in (1, 2, 4, 8):
                    sh = x.at[jnp.maximum(lane - d, 0)].get(
                        mode="promise_in_bounds")
                    x = x + jnp.where(lane >= d, sh, 0)
                t = lane + 1 + skip
                lo = jnp.zeros((16,), jnp.int32)
                for s_ in (8, 4, 2, 1):
                    xm = x.at[lo + (s_ - 1)].get(mode="promise_in_bounds")
                    lo = lo + jnp.where(xm < t, s_, 0)
                comp = (gbase + 16 * g) + lo
                a = lax.bitwise_and(off, 15)
                off16 = off - a
                rot = comp.at[lax.bitwise_and(lane - a, 15)].get(
                    mode="promise_in_bounds")
                prev = idxbuf[pl.ds(off16, 16)]
                idxbuf[pl.ds(off16, 16)] = jnp.where(lane >= a, rot, prev)
                idxbuf[pl.ds(off16 + 16, 16)] = rot
                return 0

            lax.fori_loop(0, NG, step, 0)
        return 0
    lax.fori_loop(0, NB, blk_body, 0)

    ka = lax.bitwise_and(kend, 15)
    k16 = kend - ka
    prev = idxbuf[pl.ds(k16, 16)]
    idxbuf[pl.ds(k16, 16)] = jnp.where(lane < ka, prev, dummy16)
    idxbuf[pl.ds(k16 + 16, 16)] = dummy16

    def superstep(s, _):
        @pl.when(s * (SUPER * GROUP) < kend)
        def _():
            def fire(g, _):
                pltpu.async_copy(
                    table_hbm.at[idxbuf.at[pl.ds((s * SUPER + g) * GROUP,
                                                 GROUP)]],
                    rowbuf.at[pl.ds(g * GROUP, GROUP)],
                    gsem)
                return 0
            lax.fori_loop(0, SUPER, fire, 0)

            def drain(g, _):
                pltpu.make_async_copy(
                    table_hbm.at[idxbuf.at[pl.ds(0, GROUP)]],
                    rowbuf.at[pl.ds(g * GROUP, GROUP)],
                    gsem,
                ).wait()
                return 0
            lax.fori_loop(0, SUPER, drain, 0)

            def repack(r, _):
                pack32[r, pl.ds(0, 16)] = rowbuf[r, pl.ds(0, 16)]
                pack32[r, pl.ds(16, 16)] = rowbuf[r, pl.ds(16, 16)]
                return 0
            lax.fori_loop(0, SUPER * GROUP, repack, 0)

            pltpu.sync_copy(
                pack32,
                out_hbm.at[pl.ds(base + s * (SUPER * GROUP),
                                 SUPER * GROUP)],
            )
        return 0
    lax.fori_loop(0, NSUP, superstep, 0)

    ones16 = jnp.ones((16,), jnp.float32)

    def ones_fill(r, _):
        pack32[r, pl.ds(0, 16)] = ones16
        pack32[r, pl.ds(16, 16)] = ones16
        return 0
    lax.fori_loop(0, SUPER * GROUP, ones_fill, 0)

    def ones_step(s, _):
        @pl.when(s * (SUPER * GROUP) >= kend)
        def _():
            pltpu.sync_copy(
                pack32,
                out_hbm.at[pl.ds(base + s * (SUPER * GROUP),
                                 SUPER * GROUP)],
            )
        return 0
    lax.fori_loop(0, NSUP, ones_step, 0)


def make_sc_call(interpret=False):
    mesh = plsc.VectorSubcoreMesh(core_axis_name="c", subcore_axis_name="s")
    return functools.partial(
        pl.kernel,
        out_type=jax.ShapeDtypeStruct((N, OPAD), jnp.float32),
        mesh=mesh,
        scratch_types=[
            pltpu.SMEM((NB + 16,), jnp.int32),
            pltpu.SMEM((NG,), jnp.int32),
            pltpu.VMEM((BLK,), jnp.int32),
            pltpu.VMEM((NG,), jnp.int32),
            pltpu.VMEM((CHUNK + 32,), jnp.int32),
            pltpu.VMEM((SUPER * GROUP, TPAD), jnp.float32),
            pltpu.VMEM((SUPER * GROUP, OPAD), jnp.float32),
            pltpu.SemaphoreType.DMA,
        ],
        interpret=interpret,
    )(sc_compact_body)


def kernel(logits, label, temperature):
    del label
    l3 = logits.reshape(BATCH, C, PIX)
    table3, mask3, pref3, gpref4 = make_tc_call()(temperature, l3)
    table2 = table3.reshape((NB + 1) * BLK, TPAD)
    mask1 = mask3[:NB].reshape(N)
    prefix1 = jnp.pad(pref3.reshape(NB + 1), (0, 15))
    gpref1 = gpref4.reshape((NB + 1) * NG)
    outp = make_sc_call()(table2, mask1, prefix1, gpref1)
    return outp[:, :C]


# speedup vs baseline: 1.0848x; 1.0848x over previous
import functools

import jax
import jax.numpy as jnp
from jax import lax
from jax.experimental import pallas as pl
from jax.experimental.pallas import tpu as pltpu
from jax.experimental.pallas import tpu_sc as plsc

BATCH = 4
C = 19
TPAD = 128
PIX = 512 * 512
N = BATCH * PIX
THR = 2.3

BLK = 8192
NB = N // BLK
NBB = PIX // BLK
NG = BLK // 16

NW = 32
CHUNK = N // NW
GROUP = 128
NGRP = CHUNK // GROUP
SUPER = 4
NSUP = NGRP // SUPER
DUMMY = N


def _tree_sum(vals):
    vals = list(vals)
    n = 16
    while n >= 1:
        for i in range(n):
            if i + n < len(vals):
                vals[i] = vals[i] + vals[i + n]
        vals = vals[:n]
        n //= 2
    return vals[0]


def _tree_max(vals):
    vals = list(vals)
    n = 16
    while n >= 1:
        for i in range(n):
            if i + n < len(vals):
                vals[i] = jnp.maximum(vals[i], vals[i + n])
        vals = vals[:n]
        n //= 2
    return vals[0]


def tc_entropy_body(t_ref, lo_ref, table_ref, mask_ref, pref_ref, gpref_ref,
                    acc_ref):
    i = pl.program_id(0)

    @pl.when(i == 0)
    def _():
        acc_ref[0] = 0

    x = lo_ref[0]
    planes = [x[c] for c in range(C)]
    m = _tree_max(planes)
    e = jnp.exp(x - m[None, :])
    s = _tree_sum([e[c] for c in range(C)])
    p = e / s[None, :]
    t = (-p) * jnp.log(p)
    h = _tree_sum([t[c] for c in range(C)])
    msk = (h < THR).astype(jnp.int32)
    scaled = x / t_ref[0]

    pref_ref[0] = acc_ref[0].reshape(1, 1)

    mf = (h < THR).astype(jnp.float32).reshape(64, 128)
    li = lax.broadcasted_iota(jnp.int32, (128, 8), 0)
    lj = lax.broadcasted_iota(jnp.int32, (128, 8), 1)
    P = jnp.where(li // 16 == lj, 1.0, 0.0)
    g2 = jnp.dot(mf, P, preferred_element_type=jnp.float32)
    t8r = lax.broadcasted_iota(jnp.int32, (8, 8), 0)
    t8c = lax.broadcasted_iota(jnp.int32, (8, 8), 1)
    T8 = jnp.where(t8r < t8c, 1.0, 0.0)
    g2ex = jnp.dot(g2, T8, preferred_element_type=jnp.float32)
    rt = jnp.sum(g2, axis=1, keepdims=True)
    lr = lax.broadcasted_iota(jnp.int32, (64, 64), 0)
    lc = lax.broadcasted_iota(jnp.int32, (64, 64), 1)
    L = jnp.where(lc < lr, 1.0, 0.0)
    ro2 = jnp.dot(L, rt, preferred_element_type=jnp.float32)
    accf = acc_ref[0].astype(jnp.float32)
    gpref_ref[0] = (g2ex + ro2 + accf).astype(jnp.int32)

    @pl.when(i < NB)
    def _():
        table_ref[0] = jnp.concatenate(
            [scaled.T, jnp.ones((BLK, TPAD - C), jnp.float32)], axis=1)
        mask_ref[0, 0] = msk
        acc_ref[0] = acc_ref[0] + jnp.sum(msk)

    @pl.when(i == NB)
    def _():
        table_ref[0] = jnp.ones((BLK, TPAD), jnp.float32)
        mask_ref[0, 0] = jnp.zeros((BLK,), jnp.int32)


def make_tc_call(interpret=False):
    return pl.pallas_call(
        tc_entropy_body,
        grid=(NB + 1,),
        in_specs=[
            pl.BlockSpec(memory_space=pltpu.SMEM),
            pl.BlockSpec(
                (1, C, BLK),
                lambda i: (jnp.minimum(i, NB - 1) // NBB, 0,
                           jnp.minimum(i, NB - 1) % NBB),
            ),
        ],
        out_specs=[
            pl.BlockSpec((1, BLK, TPAD), lambda i: (i, 0, 0)),
            pl.BlockSpec((1, 1, BLK), lambda i: (i, 0, 0)),
            pl.BlockSpec((1, 1, 1), lambda i: (i, 0, 0)),
            pl.BlockSpec((1, 64, 8), lambda i: (i, 0, 0)),
        ],
        out_shape=[
            jax.ShapeDtypeStruct((NB + 1, BLK, TPAD), jnp.float32),
            jax.ShapeDtypeStruct((NB + 1, 1, BLK), jnp.int32),
            jax.ShapeDtypeStruct((NB + 1, 1, 1), jnp.int32),
            jax.ShapeDtypeStruct((NB + 1, 64, 8), jnp.int32),
        ],
        scratch_shapes=[pltpu.SMEM((1,), jnp.int32)],
        interpret=interpret,
    )


def sc_compact_body(table_hbm, mask_hbm, prefix_hbm, gpref_hbm, out_hbm,
                    prefsmem, gprefsmem, maskbuf, gprefv, idxbuf, rowbuf,
                    gsem):
    wid = lax.axis_index("s") * 2 + lax.axis_index("c")
    base = wid * CHUNK
    lane = lax.iota(jnp.int32, 16)
    dummy16 = jnp.full((16,), DUMMY, jnp.int32)

    pltpu.sync_copy(prefix_hbm, maskbuf.at[pl.ds(0, NB + 16)])
    for k in range((NB + 16) // 16):
        v = maskbuf[pl.ds(16 * k, 16)]
        for j in range(16):
            prefsmem[16 * k + j] = v[j]

    def prefill(i, _):
        idxbuf[pl.ds(16 * i, 16)] = dummy16
        return 0
    lax.fori_loop(0, (CHUNK + 32) // 16, prefill, 0)

    ktot = prefsmem[NB]
    kend = jnp.clip(ktot - base, 0, CHUNK)

    def blk_body(b, _):
        pb = prefsmem[b]
        pb1 = prefsmem[b + 1]
        live = jnp.logical_and(
            jnp.logical_and(pb1 > base, pb < base + CHUNK),
            pb1 > pb)

        @pl.when(live)
        def _():
            pltpu.sync_copy(mask_hbm.at[pl.ds(b * BLK, BLK)], maskbuf)
            pltpu.sync_copy(gpref_hbm.at[pl.ds(b * NG, NG)], gprefv)
            for k in range(NG // 16):
                v = gprefv[pl.ds(16 * k, 16)]
                for j in range(16):
                    gprefsmem[16 * k + j] = v[j]
            gbase = b * BLK

            def step(g, _):
                wp = gprefsmem[g] - base
                off = jnp.clip(wp, 0, kend)
                skip = off - wp
                mv = maskbuf[pl.ds(16 * g, 16)]
                mb = mv > 0
                one = jnp.where(mb, 1, 0)
                x = one
                for d in (1, 2, 4, 8):
                    sh = x.at[jnp.maximum(lane - d, 0)].get(
                        mode="promise_in_bounds")
                    x = x + jnp.where(lane >= d, sh, 0)
                t = lane + 1 + skip
                lo = jnp.zeros((16,), jnp.int32)
                for s_ in (8, 4, 2, 1):
                    xm = x.at[lo + (s_ - 1)].get(mode="promise_in_bounds")
                    lo = lo + jnp.where(xm < t, s_, 0)
                comp = (gbase + 16 * g) + lo
                a = lax.bitwise_and(off, 15)
                off16 = off - a
                rot = comp.at[lax.bitwise_and(lane - a, 15)].get(
                    mode="promise_in_bounds")
                prev = idxbuf[pl.ds(off16, 16)]
                idxbuf[pl.ds(off16, 16)] = jnp.where(lane >= a, rot, prev)
                idxbuf[pl.ds(off16 + 16, 16)] = rot
                return 0

            lax.fori_loop(0, NG, step, 0)
        return 0
    lax.fori_loop(0, NB, blk_body, 0)

    ka = lax.bitwise_and(kend, 15)
    k16 = kend - ka
    prev = idxbuf[pl.ds(k16, 16)]
    idxbuf[pl.ds(k16, 16)] = jnp.where(lane < ka, prev, dummy16)
    idxbuf[pl.ds(k16 + 16, 16)] = dummy16

    def superstep(s, _):
        @pl.when(s * (SUPER * GROUP) < kend)
        def _():
            def fire(g, _):
                pltpu.async_copy(
                    table_hbm.at[idxbuf.at[pl.ds((s * SUPER + g) * GROUP,
                                                 GROUP)]],
                    rowbuf.at[pl.ds(g * GROUP, GROUP)],
                    gsem)
                return 0
            lax.fori_loop(0, SUPER, fire, 0)

            def drain(g, _):
                pltpu.make_async_copy(
                    table_hbm.at[idxbuf.at[pl.ds(0, GROUP)]],
                    rowbuf.at[pl.ds(g * GROUP, GROUP)],
                    gsem,
                ).wait()
                return 0
            lax.fori_loop(0, SUPER, drain, 0)

            pltpu.sync_copy(
                rowbuf,
                out_hbm.at[pl.ds(base + s * (SUPER * GROUP),
                                 SUPER * GROUP)],
            )
        return 0
    lax.fori_loop(0, NSUP, superstep, 0)

    pltpu.sync_copy(
        table_hbm.at[pl.ds(NB * BLK, SUPER * GROUP)], rowbuf)

    def ones_step(s, _):
        @pl.when(s * (SUPER * GROUP) >= kend)
        def _():
            pltpu.sync_copy(
                rowbuf,
                out_hbm.at[pl.ds(base + s * (SUPER * GROUP),
                                 SUPER * GROUP)],
            )
        return 0
    lax.fori_loop(0, NSUP, ones_step, 0)


def make_sc_call(interpret=False):
    mesh = plsc.VectorSubcoreMesh(core_axis_name="c", subcore_axis_name="s")
    return functools.partial(
        pl.kernel,
        out_type=jax.ShapeDtypeStruct((N, TPAD), jnp.float32),
        mesh=mesh,
        scratch_types=[
            pltpu.SMEM((NB + 16,), jnp.int32),
            pltpu.SMEM((NG,), jnp.int32),
            pltpu.VMEM((BLK,), jnp.int32),
            pltpu.VMEM((NG,), jnp.int32),
            pltpu.VMEM((CHUNK + 32,), jnp.int32),
            pltpu.VMEM((SUPER * GROUP, TPAD), jnp.float32),
            pltpu.SemaphoreType.DMA,
        ],
        interpret=interpret,
    )(sc_compact_body)


def kernel(logits, label, temperature):
    del label
    l3 = logits.reshape(BATCH, C, PIX)
    table3, mask3, pref3, gpref4 = make_tc_call()(temperature, l3)
    table2 = table3.reshape((NB + 1) * BLK, TPAD)
    mask1 = mask3[:NB].reshape(N)
    prefix1 = jnp.pad(pref3.reshape(NB + 1), (0, 15))
    gpref1 = gpref4.reshape((NB + 1) * NG)
    outp = make_sc_call()(table2, mask1, prefix1, gpref1)
    return outp[:, :C]
